# baseline (speedup 1.0000x reference)
import jax
import jax.numpy as jnp
from jax.experimental import pallas as pl
from jax.experimental.pallas import tpu as pltpu

_R, _C = 1_000_000, 64
_BR = 25_000
_NBLK = _R // _BR


def _copy_body(a_ref, o_ref):
    o_ref[...] = a_ref[...]

    @pl.when(pl.program_id(0) == 0)
    def _apply_scatter():
        r = jax.lax.broadcasted_iota(jnp.int32, (8, _C), 0)
        c = jax.lax.broadcasted_iota(jnp.int32, (8, _C), 1)
        upd = jnp.where((r == 0) & (c == 0), 1.0, 0.0) + jnp.where(
            (r == 1) & (c == 0), 2.0, 0.0
        )
        o_ref[0:8, :] += upd.astype(o_ref.dtype)


def kernel(A):
    return pl.pallas_call(
        _copy_body,
        grid=(_NBLK,),
        in_specs=[pl.BlockSpec((_BR, _C), lambda i: (i, 0))],
        out_specs=pl.BlockSpec((_BR, _C), lambda i: (i, 0)),
        out_shape=jax.ShapeDtypeStruct((_R, _C), A.dtype),
        compiler_params=pltpu.CompilerParams(
            dimension_semantics=("arbitrary",),
        ),
    )(A)
